# baseline (speedup 1.0000x reference)
import functools

import jax
import jax.numpy as jnp
from jax import lax
from jax.experimental import pallas as pl
from jax.experimental.pallas import tpu as pltpu
from jax.experimental.pallas import tpu_sc as plsc

N_NODES = 10000
N_EDGES = 320000
D = 128
N_GRAPHS = 256

NC, NS = 2, 16
CHUNK = 80
EDGES_PER_TILE = N_EDGES // NS
NCHUNK = EDGES_PER_TILE // CHUNK
EPT_PAD = NCHUNK * CHUNK
N_PAD = 10240
HALF = N_PAD // 2
ACC_ROWS = HALF + 8
ZROWS = HALF // NS
NBUF = 2

TC_BLK = 1000
TC_NB = N_NODES // TC_BLK


def _sc_agg_body(src_hbm, dst_hbm, x_hbm, zeros_hbm, out_hbm,
                 src_v, dst_v, rows_v, acc_sh, gsem, ssem):
    c = lax.axis_index("c")
    s = lax.axis_index("s")
    lo = c * HALF

    pltpu.sync_copy(zeros_hbm, acc_sh.at[pl.ds(s * ZROWS, ZROWS)])

    pltpu.sync_copy(src_hbm.at[s], src_v)
    pltpu.sync_copy(dst_hbm.at[s], dst_v)

    plsc.subcore_barrier()

    pltpu.async_copy(x_hbm.at[src_v.at[0]], rows_v.at[0], gsem)

    def body(i, _):
        b = lax.rem(i, NBUF)

        for j in range(CHUNK // 16):
            v = dst_v[i, pl.ds(j * 16, 16)] - lo
            ok = (v >= 0) & (v < HALF)
            dst_v[i, pl.ds(j * 16, 16)] = jnp.where(ok, v, HALF)

        pltpu.make_async_copy(x_hbm.at[src_v.at[i]], rows_v.at[b], gsem).wait()

        pltpu.async_copy(rows_v.at[b], acc_sh.at[dst_v.at[i]], ssem, add=True)

        @pl.when(i >= 1)
        def _():
            bp = lax.rem(i - 1, NBUF)
            pltpu.make_async_copy(rows_v.at[bp],
                                  acc_sh.at[dst_v.at[i - 1]], ssem).wait()

        @pl.when(i + 1 < NCHUNK)
        def _():
            pltpu.async_copy(x_hbm.at[src_v.at[i + 1]],
                             rows_v.at[lax.rem(i + 1, NBUF)], gsem)
        return 0

    lax.fori_loop(0, NCHUNK, body, 0)

    bl = lax.rem(NCHUNK - 1, NBUF)
    pltpu.make_async_copy(rows_v.at[bl],
                          acc_sh.at[dst_v.at[NCHUNK - 1]], ssem).wait()

    plsc.subcore_barrier()

    pltpu.sync_copy(acc_sh.at[pl.ds(s * ZROWS, ZROWS)],
                    out_hbm.at[c, pl.ds(s * ZROWS, ZROWS)])


@functools.cache
def _sc_agg():
    return functools.partial(
        pl.kernel,
        out_type=jax.ShapeDtypeStruct((NC, HALF, D), jnp.float32),
        mesh=plsc.VectorSubcoreMesh(core_axis_name="c", subcore_axis_name="s",
                                    num_cores=NC, num_subcores=NS),
        scratch_types=[
            pltpu.VMEM((NCHUNK, CHUNK), jnp.int32),
            pltpu.VMEM((NCHUNK, CHUNK), jnp.int32),
            pltpu.VMEM((NBUF, CHUNK, D), jnp.float32),
            pltpu.VMEM_SHARED((ACC_ROWS, D), jnp.float32),
            pltpu.SemaphoreType.DMA,
            pltpu.SemaphoreType.DMA,
        ],
    )(_sc_agg_body)


def _tc_body(x_ref, a_ref, b_ref, Wm_ref, bm_ref,
             W1_ref, b1_ref, W2_ref, b2_ref, out_ref, sums_ref, cnts_ref):
    i = pl.program_id(0)

    @pl.when(i == 0)
    def _():
        sums_ref[:] = jnp.zeros_like(sums_ref)
        cnts_ref[:] = jnp.zeros_like(cnts_ref)

    xa = x_ref[:] + a_ref[:]
    h = jnp.maximum(
        jnp.dot(xa, Wm_ref[:], preferred_element_type=jnp.float32) + bm_ref[:],
        0.0)

    bid = b_ref[0, 0, :]
    gids = lax.broadcasted_iota(jnp.int32, (N_GRAPHS, TC_BLK), 0)
    onehot = jnp.where(gids == bid[None, :], 1.0, 0.0)
    sums_ref[:] += jnp.dot(onehot, h, preferred_element_type=jnp.float32)
    cnts_ref[:] += jnp.broadcast_to(
        jnp.sum(onehot, axis=1, keepdims=True), (N_GRAPHS, D))

    @pl.when(i == TC_NB - 1)
    def _():
        H = sums_ref[:] / jnp.maximum(cnts_ref[:], 1.0)
        Z1 = jnp.maximum(
            jnp.dot(H, W1_ref[:], preferred_element_type=jnp.float32) + b1_ref[:],
            0.0)
        Z = jnp.dot(Z1, W2_ref[:], preferred_element_type=jnp.float32) + b2_ref[:]
        out_ref[:] = Z + H


_tc_call = pl.pallas_call(
    _tc_body,
    grid=(TC_NB,),
    in_specs=[
        pl.BlockSpec((TC_BLK, D), lambda i: (i, 0)),
        pl.BlockSpec((TC_BLK, D), lambda i: (i, 0)),
        pl.BlockSpec((1, 1, TC_BLK), lambda i: (i, 0, 0)),
        pl.BlockSpec((D, D), lambda i: (0, 0)),
        pl.BlockSpec((1, D), lambda i: (0, 0)),
        pl.BlockSpec((D, D), lambda i: (0, 0)),
        pl.BlockSpec((1, D), lambda i: (0, 0)),
        pl.BlockSpec((D, D), lambda i: (0, 0)),
        pl.BlockSpec((1, D), lambda i: (0, 0)),
    ],
    out_specs=pl.BlockSpec((N_GRAPHS, D), lambda i: (0, 0)),
    out_shape=jax.ShapeDtypeStruct((N_GRAPHS, D), jnp.float32),
    scratch_shapes=[
        pltpu.VMEM((N_GRAPHS, D), jnp.float32),
        pltpu.VMEM((N_GRAPHS, D), jnp.float32),
    ],
    compiler_params=pltpu.CompilerParams(
        dimension_semantics=("arbitrary",)),
)


def kernel(x, edge_index, batch, W_msg, b_msg, W1, b1, W2, b2):
    src = edge_index[0].astype(jnp.int32).reshape(NS, NCHUNK, CHUNK)
    dst = edge_index[1].astype(jnp.int32).reshape(NS, NCHUNK, CHUNK)
    zeros = jnp.zeros((ZROWS, D), jnp.float32)

    agg = _sc_agg()(src, dst, x, zeros).reshape(N_PAD, D)

    batch3 = batch.astype(jnp.int32).reshape(TC_NB, 1, TC_BLK)
    return _tc_call(x, agg, batch3,
                    W_msg, b_msg.reshape(1, D), W1, b1.reshape(1, D),
                    W2, b2.reshape(1, D))


# speedup vs baseline: 1.0156x; 1.0156x over previous
import functools

import jax
import jax.numpy as jnp
from jax import lax
from jax.experimental import pallas as pl
from jax.experimental.pallas import tpu as pltpu
from jax.experimental.pallas import tpu_sc as plsc

N_NODES = 10000
N_EDGES = 320000
D = 128
N_GRAPHS = 256

NC, NS = 2, 16
CHUNK = 80
EDGES_PER_TILE = N_EDGES // NS
NCHUNK = EDGES_PER_TILE // CHUNK
EPT_PAD = NCHUNK * CHUNK
N_PAD = 10240
HALF = N_PAD // 2
ACC_ROWS = HALF + 8
ZROWS = HALF // NS
NBUF = 2

TC_BLK = 2000
TC_NB = N_NODES // TC_BLK


def _sc_agg_body(src_hbm, dst_hbm, x_hbm, zeros_hbm, out_hbm,
                 src_v, dst_v, rows_v, acc_sh, gsem):
    c = lax.axis_index("c")
    s = lax.axis_index("s")
    lo = c * HALF

    pltpu.sync_copy(zeros_hbm, acc_sh.at[pl.ds(s * ZROWS, ZROWS)])

    pltpu.sync_copy(src_hbm.at[s], src_v)
    pltpu.sync_copy(dst_hbm.at[s], dst_v)

    plsc.subcore_barrier()

    pltpu.async_copy(x_hbm.at[src_v.at[0]], rows_v.at[0], gsem)

    def body(i, _):
        b = lax.rem(i, NBUF)

        for j in range(CHUNK // 16):
            v = dst_v[i, pl.ds(j * 16, 16)] - lo
            ok = (v >= 0) & (v < HALF)
            dst_v[i, pl.ds(j * 16, 16)] = jnp.where(ok, v, HALF)

        pltpu.make_async_copy(x_hbm.at[src_v.at[i]], rows_v.at[b], gsem).wait()

        @pl.when(i + 1 < NCHUNK)
        def _():
            pltpu.async_copy(x_hbm.at[src_v.at[i + 1]],
                             rows_v.at[lax.rem(i + 1, NBUF)], gsem)

        pltpu.sync_copy(rows_v.at[b], acc_sh.at[dst_v.at[i]], add=True)
        return 0

    lax.fori_loop(0, NCHUNK, body, 0)

    plsc.subcore_barrier()

    pltpu.sync_copy(acc_sh.at[pl.ds(s * ZROWS, ZROWS)],
                    out_hbm.at[c, pl.ds(s * ZROWS, ZROWS)])


@functools.cache
def _sc_agg():
    return functools.partial(
        pl.kernel,
        out_type=jax.ShapeDtypeStruct((NC, HALF, D), jnp.float32),
        mesh=plsc.VectorSubcoreMesh(core_axis_name="c", subcore_axis_name="s",
                                    num_cores=NC, num_subcores=NS),
        scratch_types=[
            pltpu.VMEM((NCHUNK, CHUNK), jnp.int32),
            pltpu.VMEM((NCHUNK, CHUNK), jnp.int32),
            pltpu.VMEM((NBUF, CHUNK, D), jnp.float32),
            pltpu.VMEM_SHARED((ACC_ROWS, D), jnp.float32),
            pltpu.SemaphoreType.DMA,
        ],
    )(_sc_agg_body)


def _tc_body(x_ref, a_ref, b_ref, Wm_ref, bm_ref,
             W1_ref, b1_ref, W2_ref, b2_ref, out_ref, sums_ref, cnts_ref):
    i = pl.program_id(0)

    @pl.when(i == 0)
    def _():
        sums_ref[:] = jnp.zeros_like(sums_ref)
        cnts_ref[:] = jnp.zeros_like(cnts_ref)

    xa = x_ref[:] + a_ref[:]
    h = jnp.maximum(
        jnp.dot(xa, Wm_ref[:], preferred_element_type=jnp.float32) + bm_ref[:],
        0.0)

    bid = b_ref[0, 0, :]
    gids = lax.broadcasted_iota(jnp.int32, (N_GRAPHS, TC_BLK), 0)
    onehot = jnp.where(gids == bid[None, :], 1.0, 0.0)
    sums_ref[:] += jnp.dot(onehot, h, preferred_element_type=jnp.float32)
    cnts_ref[:] += jnp.broadcast_to(
        jnp.sum(onehot, axis=1, keepdims=True), (N_GRAPHS, D))

    @pl.when(i == TC_NB - 1)
    def _():
        H = sums_ref[:] / jnp.maximum(cnts_ref[:], 1.0)
        Z1 = jnp.maximum(
            jnp.dot(H, W1_ref[:], preferred_element_type=jnp.float32) + b1_ref[:],
            0.0)
        Z = jnp.dot(Z1, W2_ref[:], preferred_element_type=jnp.float32) + b2_ref[:]
        out_ref[:] = Z + H


_tc_call = pl.pallas_call(
    _tc_body,
    grid=(TC_NB,),
    in_specs=[
        pl.BlockSpec((TC_BLK, D), lambda i: (i, 0)),
        pl.BlockSpec((TC_BLK, D), lambda i: (i, 0)),
        pl.BlockSpec((1, 1, TC_BLK), lambda i: (i, 0, 0)),
        pl.BlockSpec((D, D), lambda i: (0, 0)),
        pl.BlockSpec((1, D), lambda i: (0, 0)),
        pl.BlockSpec((D, D), lambda i: (0, 0)),
        pl.BlockSpec((1, D), lambda i: (0, 0)),
        pl.BlockSpec((D, D), lambda i: (0, 0)),
        pl.BlockSpec((1, D), lambda i: (0, 0)),
    ],
    out_specs=pl.BlockSpec((N_GRAPHS, D), lambda i: (0, 0)),
    out_shape=jax.ShapeDtypeStruct((N_GRAPHS, D), jnp.float32),
    scratch_shapes=[
        pltpu.VMEM((N_GRAPHS, D), jnp.float32),
        pltpu.VMEM((N_GRAPHS, D), jnp.float32),
    ],
    compiler_params=pltpu.CompilerParams(
        dimension_semantics=("arbitrary",)),
)


def kernel(x, edge_index, batch, W_msg, b_msg, W1, b1, W2, b2):
    src = edge_index[0].astype(jnp.int32).reshape(NS, NCHUNK, CHUNK)
    dst = edge_index[1].astype(jnp.int32).reshape(NS, NCHUNK, CHUNK)
    zeros = jnp.zeros((ZROWS, D), jnp.float32)

    agg = _sc_agg()(src, dst, x, zeros).reshape(N_PAD, D)

    batch3 = batch.astype(jnp.int32).reshape(TC_NB, 1, TC_BLK)
    return _tc_call(x, agg, batch3,
                    W_msg, b_msg.reshape(1, D), W1, b1.reshape(1, D),
                    W2, b2.reshape(1, D))


# speedup vs baseline: 1.0230x; 1.0073x over previous
import functools

import jax
import jax.numpy as jnp
from jax import lax
from jax.experimental import pallas as pl
from jax.experimental.pallas import tpu as pltpu
from jax.experimental.pallas import tpu_sc as plsc

N_NODES = 10000
N_EDGES = 320000
D = 128
N_GRAPHS = 256

NC, NS = 2, 16
CHUNK = 80
EDGES_PER_TILE = N_EDGES // NS
NCHUNK = EDGES_PER_TILE // CHUNK
EPT_PAD = NCHUNK * CHUNK
N_PAD = 10240
HALF = N_PAD // 2
ACC_ROWS = HALF + 8
ZROWS = HALF // NS
NBUF = 2

TC_BLK = 2000
TC_NB = N_NODES // TC_BLK


def _sc_agg_body(src_hbm, dst_hbm, x_hbm, zeros_hbm, out_hbm,
                 src_v, dst_v, rows_v, acc_sh, gsem, isem):
    c = lax.axis_index("c")
    s = lax.axis_index("s")
    lo = c * HALF

    pltpu.sync_copy(src_hbm.at[s], src_v)
    pltpu.async_copy(zeros_hbm, acc_sh.at[pl.ds(s * ZROWS, ZROWS)], isem)
    pltpu.async_copy(dst_hbm.at[s], dst_v, isem)

    pltpu.async_copy(x_hbm.at[src_v.at[0]], rows_v.at[0], gsem)

    pltpu.make_async_copy(zeros_hbm,
                          acc_sh.at[pl.ds(s * ZROWS, ZROWS)], isem).wait()
    pltpu.make_async_copy(dst_hbm.at[s], dst_v, isem).wait()

    plsc.subcore_barrier()

    def body(i, _):
        b = lax.rem(i, NBUF)

        for j in range(CHUNK // 16):
            v = dst_v[i, pl.ds(j * 16, 16)] - lo
            ok = (v >= 0) & (v < HALF)
            dst_v[i, pl.ds(j * 16, 16)] = jnp.where(ok, v, HALF)

        pltpu.make_async_copy(x_hbm.at[src_v.at[i]], rows_v.at[b], gsem).wait()

        @pl.when(i + 1 < NCHUNK)
        def _():
            pltpu.async_copy(x_hbm.at[src_v.at[i + 1]],
                             rows_v.at[lax.rem(i + 1, NBUF)], gsem)

        pltpu.sync_copy(rows_v.at[b], acc_sh.at[dst_v.at[i]], add=True)
        return 0

    lax.fori_loop(0, NCHUNK, body, 0)

    plsc.subcore_barrier()

    pltpu.sync_copy(acc_sh.at[pl.ds(s * ZROWS, ZROWS)],
                    out_hbm.at[c, pl.ds(s * ZROWS, ZROWS)])


@functools.cache
def _sc_agg():
    return functools.partial(
        pl.kernel,
        out_type=jax.ShapeDtypeStruct((NC, HALF, D), jnp.float32),
        mesh=plsc.VectorSubcoreMesh(core_axis_name="c", subcore_axis_name="s",
                                    num_cores=NC, num_subcores=NS),
        scratch_types=[
            pltpu.VMEM((NCHUNK, CHUNK), jnp.int32),
            pltpu.VMEM((NCHUNK, CHUNK), jnp.int32),
            pltpu.VMEM((NBUF, CHUNK, D), jnp.float32),
            pltpu.VMEM_SHARED((ACC_ROWS, D), jnp.float32),
            pltpu.SemaphoreType.DMA,
            pltpu.SemaphoreType.DMA,
        ],
    )(_sc_agg_body)


def _tc_body(x_ref, a_ref, b_ref, Wm_ref, bm_ref,
             W1_ref, b1_ref, W2_ref, b2_ref, out_ref, sums_ref, cnts_ref):
    i = pl.program_id(0)

    @pl.when(i == 0)
    def _():
        sums_ref[:] = jnp.zeros_like(sums_ref)
        cnts_ref[:] = jnp.zeros_like(cnts_ref)

    xa = x_ref[:] + a_ref[:]
    h = jnp.maximum(
        jnp.dot(xa, Wm_ref[:], preferred_element_type=jnp.float32) + bm_ref[:],
        0.0)

    bid = b_ref[0, 0, :]
    gids = lax.broadcasted_iota(jnp.int32, (N_GRAPHS, TC_BLK), 0)
    onehot = jnp.where(gids == bid[None, :], 1.0, 0.0)
    sums_ref[:] += jnp.dot(onehot, h, preferred_element_type=jnp.float32)
    cnts_ref[:] += jnp.broadcast_to(
        jnp.sum(onehot, axis=1, keepdims=True), (N_GRAPHS, D))

    @pl.when(i == TC_NB - 1)
    def _():
        H = sums_ref[:] / jnp.maximum(cnts_ref[:], 1.0)
        Z1 = jnp.maximum(
            jnp.dot(H, W1_ref[:], preferred_element_type=jnp.float32) + b1_ref[:],
            0.0)
        Z = jnp.dot(Z1, W2_ref[:], preferred_element_type=jnp.float32) + b2_ref[:]
        out_ref[:] = Z + H


_tc_call = pl.pallas_call(
    _tc_body,
    grid=(TC_NB,),
    in_specs=[
        pl.BlockSpec((TC_BLK, D), lambda i: (i, 0)),
        pl.BlockSpec((TC_BLK, D), lambda i: (i, 0)),
        pl.BlockSpec((1, 1, TC_BLK), lambda i: (i, 0, 0)),
        pl.BlockSpec((D, D), lambda i: (0, 0)),
        pl.BlockSpec((1, D), lambda i: (0, 0)),
        pl.BlockSpec((D, D), lambda i: (0, 0)),
        pl.BlockSpec((1, D), lambda i: (0, 0)),
        pl.BlockSpec((D, D), lambda i: (0, 0)),
        pl.BlockSpec((1, D), lambda i: (0, 0)),
    ],
    out_specs=pl.BlockSpec((N_GRAPHS, D), lambda i: (0, 0)),
    out_shape=jax.ShapeDtypeStruct((N_GRAPHS, D), jnp.float32),
    scratch_shapes=[
        pltpu.VMEM((N_GRAPHS, D), jnp.float32),
        pltpu.VMEM((N_GRAPHS, D), jnp.float32),
    ],
    compiler_params=pltpu.CompilerParams(
        dimension_semantics=("arbitrary",)),
)


def kernel(x, edge_index, batch, W_msg, b_msg, W1, b1, W2, b2):
    src = edge_index[0].astype(jnp.int32).reshape(NS, NCHUNK, CHUNK)
    dst = edge_index[1].astype(jnp.int32).reshape(NS, NCHUNK, CHUNK)
    zeros = jnp.zeros((ZROWS, D), jnp.float32)

    agg = _sc_agg()(src, dst, x, zeros).reshape(N_PAD, D)

    batch3 = batch.astype(jnp.int32).reshape(TC_NB, 1, TC_BLK)
    return _tc_call(x, agg, batch3,
                    W_msg, b_msg.reshape(1, D), W1, b1.reshape(1, D),
                    W2, b2.reshape(1, D))
